# baseline (speedup 1.0000x reference)
import functools

import jax
import jax.numpy as jnp
from jax import lax
from jax.experimental import pallas as pl
from jax.experimental.pallas import tpu as pltpu
from jax.experimental.pallas import tpu_sc as plsc

N_SRC = 10000
N_DST = 10000
E = 160000
D = 256
DH = 128

K = 64
EPT = E // 16
CHP = 160
SB = 16
NB = 4
NPAD = 10240
RPT = NPAD // 16
HR = NPAD // 128


def _gelu(x):
    return 0.5 * x * (1.0 + lax.erf(x * 0.7071067811865476))


def _dot_t(x, w):
    return lax.dot_general(x, w, (((1,), (1,)), ((), ())),
                           preferred_element_type=jnp.float32)


def _embed_body(xs_ref, xd_ref, ws_ref, bs_ref, wd_ref, bd_ref, wl_ref,
                wr_ref, bsage_ref, glo_ref, ghi_ref, z_ref):
    h_src = _gelu(_dot_t(xs_ref[...], ws_ref[...]) + bs_ref[...])
    g = _dot_t(h_src, wl_ref[...])
    glo_ref[...] = g[:, :DH]
    ghi_ref[...] = g[:, DH:]
    h_dst = _gelu(_dot_t(xd_ref[...], wd_ref[...]) + bd_ref[...])
    z_ref[...] = _dot_t(h_dst, wr_ref[...]) + bsage_ref[...]


_EMBED_R = 2000


def _embed(x_src, x_dst, W_src, b_src, W_dst, b_dst, W_l, W_r, b_sage):
    n = N_SRC // _EMBED_R
    row = lambda i: (i, 0)
    full = lambda i: (0, 0)
    gh = jax.ShapeDtypeStruct((N_SRC, DH), jnp.float32)
    return pl.pallas_call(
        _embed_body,
        grid=(n,),
        in_specs=[
            pl.BlockSpec((_EMBED_R, D), row),
            pl.BlockSpec((_EMBED_R, D), row),
            pl.BlockSpec((D, D), full),
            pl.BlockSpec((1, D), full),
            pl.BlockSpec((D, D), full),
            pl.BlockSpec((1, D), full),
            pl.BlockSpec((D, D), full),
            pl.BlockSpec((D, D), full),
            pl.BlockSpec((1, D), full),
        ],
        out_specs=[
            pl.BlockSpec((_EMBED_R, DH), row),
            pl.BlockSpec((_EMBED_R, DH), row),
            pl.BlockSpec((_EMBED_R, D), row),
        ],
        out_shape=[gh, gh, jax.ShapeDtypeStruct((N_DST, D), jnp.float32)],
    )(x_src, x_dst, W_src, b_src, W_dst, b_dst, W_l, W_r, b_sage)


def _sc_body(g_lo, g_hi, src3d, dst3d, zrow_hbm, zcnt_hbm, iota_hbm,
             out_lo, out_hi, ocnt0, ocnt1, svm, dvm, gbufs, hist,
             iota_v, acc, cntS, sems_g, sems_s):
    c = lax.axis_index("c")
    s = lax.axis_index("s")
    row0 = s * RPT

    for i in range(RPT // 128):
        pltpu.sync_copy(zrow_hbm, acc.at[pl.ds(row0 + i * 128, 128)])
    pltpu.sync_copy(zcnt_hbm, hist)

    @pl.when(s == 0)
    def _():
        pltpu.sync_copy(zcnt_hbm, cntS)

    pltpu.sync_copy(iota_hbm, iota_v)
    plsc.subcore_barrier()

    ones_i = jnp.full((16,), 1, jnp.int32)

    def gather(jj):
        @pl.when(c == 0)
        def _():
            pltpu.async_copy(g_lo.at[svm.at[jj]], gbufs[jj % NB],
                             sems_g[jj % NB])

        @pl.when(c == 1)
        def _():
            pltpu.async_copy(g_hi.at[svm.at[jj]], gbufs[jj % NB],
                             sems_g[jj % NB])

    def gwait(jj):
        pltpu.make_async_copy(g_lo.at[svm.at[jj]], gbufs[jj % NB],
                              sems_g[jj % NB]).wait()

    def swait(jj):
        pltpu.make_async_copy(gbufs[jj % NB], acc.at[dvm.at[jj]],
                              sems_s[jj % NB]).wait()

    def block(b, carry):
        pltpu.sync_copy(src3d.at[s, pl.ds(b * SB, SB)], svm)
        pltpu.sync_copy(dst3d.at[s, pl.ds(b * SB, SB)], dvm)
        gather(0)
        gather(1)
        for jj in range(SB):
            gwait(jj)
            pltpu.async_copy(gbufs[jj % NB], acc.at[dvm.at[jj]],
                             sems_s[jj % NB], add=True)

            @pl.when(c == (jj % 2))
            def _():
                for i in range(K // 16):
                    d16 = dvm[jj, pl.ds(i * 16, 16)]
                    plsc.addupdate_scatter(
                        hist,
                        [lax.shift_right_logical(d16, 7),
                         lax.bitwise_and(d16, 127)],
                        ones_i)

            if jj >= 2:
                swait(jj - 2)
            if jj + 2 < SB:
                gather(jj + 2)

        swait(SB - 2)
        swait(SB - 1)
        return carry

    lax.fori_loop(0, CHP // SB, block, 0)
    plsc.subcore_barrier()

    pltpu.sync_copy(hist, cntS.at[iota_v], add=True)
    plsc.subcore_barrier()

    @pl.when(c == 0)
    def _():
        pltpu.sync_copy(acc.at[pl.ds(row0, RPT)], out_lo.at[pl.ds(row0, RPT)])

        @pl.when(s == 0)
        def _():
            pltpu.sync_copy(cntS, ocnt0)

    @pl.when(c == 1)
    def _():
        pltpu.sync_copy(acc.at[pl.ds(row0, RPT)], out_hi.at[pl.ds(row0, RPT)])

        @pl.when(s == 0)
        def _():
            pltpu.sync_copy(cntS, ocnt1)


_sc_segsum = functools.partial(
    pl.kernel,
    out_type=[
        jax.ShapeDtypeStruct((NPAD, DH), jnp.float32),
        jax.ShapeDtypeStruct((NPAD, DH), jnp.float32),
        jax.ShapeDtypeStruct((HR, 128), jnp.int32),
        jax.ShapeDtypeStruct((HR, 128), jnp.int32),
    ],
    mesh=plsc.VectorSubcoreMesh(core_axis_name="c", subcore_axis_name="s"),
    compiler_params=pltpu.CompilerParams(needs_layout_passes=False),
    scratch_types=[
        pltpu.VMEM((SB, K), jnp.int32),
        pltpu.VMEM((SB, K), jnp.int32),
        [pltpu.VMEM((K, DH), jnp.float32)] * NB,
        pltpu.VMEM((HR, 128), jnp.int32),
        pltpu.VMEM((HR,), jnp.int32),
        pltpu.VMEM_SHARED((NPAD, DH), jnp.float32),
        pltpu.VMEM_SHARED((HR, 128), jnp.int32),
        [pltpu.SemaphoreType.DMA] * NB,
        [pltpu.SemaphoreType.DMA] * NB,
    ],
)(_sc_body)


def _post_body(slo_ref, shi_ref, c0_ref, c1_ref, z_ref, wp_ref, bp_ref,
               out_ref):
    cnt = (c0_ref[...] + c1_ref[...]).astype(jnp.float32)
    r = 1.0 / jnp.maximum(cnt, 1.0)
    agg = jnp.concatenate([slo_ref[...], shi_ref[...]], axis=1) * r
    t = agg + z_ref[...]
    out_ref[...] = _gelu(_dot_t(t, wp_ref[...]) + bp_ref[...])


def _post(slo, shi, cnt0, cnt1, z, W_post, b_post):
    n = N_DST // _EMBED_R
    row = lambda i: (i, 0)
    full = lambda i: (0, 0)
    return pl.pallas_call(
        _post_body,
        grid=(n,),
        in_specs=[
            pl.BlockSpec((_EMBED_R, DH), row),
            pl.BlockSpec((_EMBED_R, DH), row),
            pl.BlockSpec((_EMBED_R, 1), row),
            pl.BlockSpec((_EMBED_R, 1), row),
            pl.BlockSpec((_EMBED_R, D), row),
            pl.BlockSpec((D, D), full),
            pl.BlockSpec((1, D), full),
        ],
        out_specs=pl.BlockSpec((_EMBED_R, D), row),
        out_shape=jax.ShapeDtypeStruct((N_DST, D), jnp.float32),
    )(slo, shi, cnt0, cnt1, z, W_post, b_post)


def kernel(x_src, x_dst, edge_index, W_src, b_src, W_dst, b_dst, W_l, W_r,
           b_sage, W_post, b_post):
    npad_e = CHP * K - EPT
    ei = edge_index.astype(jnp.int32).reshape(2, 16, EPT)
    pad_src = jnp.zeros((16, npad_e), jnp.int32)
    pad_dst = jnp.full((16, npad_e), NPAD - 1, jnp.int32)
    src3d = jnp.concatenate([ei[0], pad_src], axis=1).reshape(16, CHP, K)
    dst3d = jnp.concatenate([ei[1], pad_dst], axis=1).reshape(16, CHP, K)
    glo, ghi, z = _embed(x_src, x_dst, W_src, b_src.reshape(1, D), W_dst,
                         b_dst.reshape(1, D), W_l, W_r, b_sage.reshape(1, D))
    zrow = jnp.zeros((128, DH), jnp.float32)
    zcnt = jnp.zeros((HR, 128), jnp.int32)
    iota = jnp.arange(HR, dtype=jnp.int32)
    slo, shi, cnt0, cnt1 = _sc_segsum(glo, ghi, src3d, dst3d, zrow, zcnt,
                                      iota)
    c0 = cnt0.reshape(NPAD, 1)[:N_DST]
    c1 = cnt1.reshape(NPAD, 1)[:N_DST]
    return _post(slo[:N_DST], shi[:N_DST], c0, c1, z, W_post,
                 b_post.reshape(1, D))


# speedup vs baseline: 1.0151x; 1.0151x over previous
import functools

import jax
import jax.numpy as jnp
from jax import lax
from jax.experimental import pallas as pl
from jax.experimental.pallas import tpu as pltpu
from jax.experimental.pallas import tpu_sc as plsc

N_SRC = 10000
N_DST = 10000
E = 160000
D = 256
DH = 128

K = 64
EPT = E // 16
CHP = 160
SB = 16
NB = 4
NPAD = 10240
RPT = NPAD // 16
HR = NPAD // 128


def _gelu(x):
    return 0.5 * x * (1.0 + lax.erf(x * 0.7071067811865476))


def _dot_t(x, w):
    return lax.dot_general(x, w, (((1,), (1,)), ((), ())),
                           preferred_element_type=jnp.float32)


def _embed_body(xs_ref, xd_ref, ws_ref, bs_ref, wd_ref, bd_ref, wl_ref,
                wr_ref, bsage_ref, glo_ref, ghi_ref, z_ref):
    h_src = _gelu(_dot_t(xs_ref[...], ws_ref[...]) + bs_ref[...])
    g = _dot_t(h_src, wl_ref[...])
    glo_ref[...] = g[:, :DH]
    ghi_ref[...] = g[:, DH:]
    h_dst = _gelu(_dot_t(xd_ref[...], wd_ref[...]) + bd_ref[...])
    z_ref[...] = _dot_t(h_dst, wr_ref[...]) + bsage_ref[...]


_EMBED_R = 2000


def _embed(x_src, x_dst, W_src, b_src, W_dst, b_dst, W_l, W_r, b_sage):
    n = N_SRC // _EMBED_R
    row = lambda i: (i, 0)
    full = lambda i: (0, 0)
    gh = jax.ShapeDtypeStruct((N_SRC, DH), jnp.float32)
    return pl.pallas_call(
        _embed_body,
        grid=(n,),
        in_specs=[
            pl.BlockSpec((_EMBED_R, D), row),
            pl.BlockSpec((_EMBED_R, D), row),
            pl.BlockSpec((D, D), full),
            pl.BlockSpec((1, D), full),
            pl.BlockSpec((D, D), full),
            pl.BlockSpec((1, D), full),
            pl.BlockSpec((D, D), full),
            pl.BlockSpec((D, D), full),
            pl.BlockSpec((1, D), full),
        ],
        out_specs=[
            pl.BlockSpec((_EMBED_R, DH), row),
            pl.BlockSpec((_EMBED_R, DH), row),
            pl.BlockSpec((_EMBED_R, D), row),
        ],
        out_shape=[gh, gh, jax.ShapeDtypeStruct((N_DST, D), jnp.float32)],
    )(x_src, x_dst, W_src, b_src, W_dst, b_dst, W_l, W_r, b_sage)


def _sc_body(g_lo, g_hi, src3d, dst3d, zrow_hbm, zcnt_hbm, iota_hbm,
             out_lo, out_hi, ocnt0, ocnt1, svm, dvm, gbufs, hist,
             iota_v, acc, cntS, sems_g, sems_s):
    c = lax.axis_index("c")
    s = lax.axis_index("s")
    row0 = s * RPT

    for i in range(RPT // 128):
        pltpu.sync_copy(zrow_hbm, acc.at[pl.ds(row0 + i * 128, 128)])
    pltpu.sync_copy(zcnt_hbm, hist)

    @pl.when(s == 0)
    def _():
        pltpu.sync_copy(zcnt_hbm, cntS)

    pltpu.sync_copy(iota_hbm, iota_v)
    plsc.subcore_barrier()

    ones_i = jnp.full((16,), 1, jnp.int32)

    def gather(jj):
        @pl.when(c == 0)
        def _():
            pltpu.async_copy(g_lo.at[svm.at[jj]], gbufs[jj % NB],
                             sems_g[jj % NB])

        @pl.when(c == 1)
        def _():
            pltpu.async_copy(g_hi.at[svm.at[jj]], gbufs[jj % NB],
                             sems_g[jj % NB])

    def gwait(jj):
        pltpu.make_async_copy(g_lo.at[svm.at[jj]], gbufs[jj % NB],
                              sems_g[jj % NB]).wait()

    def swait(jj):
        pltpu.make_async_copy(gbufs[jj % NB], acc.at[dvm.at[jj]],
                              sems_s[jj % NB]).wait()

    def block(b, carry):
        pltpu.sync_copy(src3d.at[s, pl.ds(b * SB, SB)], svm)
        pltpu.sync_copy(dst3d.at[s, pl.ds(b * SB, SB)], dvm)
        gather(0)
        gather(1)
        for jj in range(SB):
            gwait(jj)
            pltpu.async_copy(gbufs[jj % NB], acc.at[dvm.at[jj]],
                             sems_s[jj % NB], add=True)

            @pl.when(c == (jj % 2))
            def _():
                for i in range(K // 16):
                    d16 = dvm[jj, pl.ds(i * 16, 16)]
                    plsc.addupdate_scatter(
                        hist,
                        [lax.shift_right_logical(d16, 7),
                         lax.bitwise_and(d16, 127)],
                        ones_i)

            if jj >= 2:
                swait(jj - 2)
            if jj + 2 < SB:
                gather(jj + 2)

        swait(SB - 2)
        swait(SB - 1)
        return carry

    lax.fori_loop(0, CHP // SB, block, 0)
    plsc.subcore_barrier()

    pltpu.sync_copy(hist, cntS.at[iota_v], add=True)
    plsc.subcore_barrier()

    @pl.when(c == 0)
    def _():
        pltpu.sync_copy(acc.at[pl.ds(row0, RPT)], out_lo.at[pl.ds(row0, RPT)])

        @pl.when(s == 0)
        def _():
            pltpu.sync_copy(cntS, ocnt0)

    @pl.when(c == 1)
    def _():
        pltpu.sync_copy(acc.at[pl.ds(row0, RPT)], out_hi.at[pl.ds(row0, RPT)])

        @pl.when(s == 0)
        def _():
            pltpu.sync_copy(cntS, ocnt1)


_sc_segsum = functools.partial(
    pl.kernel,
    out_type=[
        jax.ShapeDtypeStruct((NPAD, DH), jnp.float32),
        jax.ShapeDtypeStruct((NPAD, DH), jnp.float32),
        jax.ShapeDtypeStruct((HR, 128), jnp.int32),
        jax.ShapeDtypeStruct((HR, 128), jnp.int32),
    ],
    mesh=plsc.VectorSubcoreMesh(core_axis_name="c", subcore_axis_name="s"),
    compiler_params=pltpu.CompilerParams(needs_layout_passes=False),
    scratch_types=[
        pltpu.VMEM((SB, K), jnp.int32),
        pltpu.VMEM((SB, K), jnp.int32),
        [pltpu.VMEM((K, DH), jnp.float32)] * NB,
        pltpu.VMEM((HR, 128), jnp.int32),
        pltpu.VMEM((HR,), jnp.int32),
        pltpu.VMEM_SHARED((NPAD, DH), jnp.float32),
        pltpu.VMEM_SHARED((HR, 128), jnp.int32),
        [pltpu.SemaphoreType.DMA] * NB,
        [pltpu.SemaphoreType.DMA] * NB,
    ],
)(_sc_body)


def _post_body(slo_ref, shi_ref, c0_ref, c1_ref, z_ref, wp_ref, bp_ref,
               out_ref):
    cnt = (c0_ref[...] + c1_ref[...]).astype(jnp.float32)
    r = 1.0 / jnp.maximum(cnt, 1.0)
    agg = jnp.concatenate([slo_ref[...], shi_ref[...]], axis=1) * r
    t = agg + z_ref[...]
    out_ref[...] = _gelu(_dot_t(t, wp_ref[...]) + bp_ref[...])


def _post(slo, shi, cnt0, cnt1, z, W_post, b_post):
    n = N_DST // _EMBED_R
    row = lambda i: (i, 0)
    full = lambda i: (0, 0)
    return pl.pallas_call(
        _post_body,
        grid=(n,),
        in_specs=[
            pl.BlockSpec((_EMBED_R, DH), row),
            pl.BlockSpec((_EMBED_R, DH), row),
            pl.BlockSpec((_EMBED_R, 1), row),
            pl.BlockSpec((_EMBED_R, 1), row),
            pl.BlockSpec((_EMBED_R, D), row),
            pl.BlockSpec((D, D), full),
            pl.BlockSpec((1, D), full),
        ],
        out_specs=pl.BlockSpec((_EMBED_R, D), row),
        out_shape=jax.ShapeDtypeStruct((N_DST, D), jnp.float32),
    )(slo, shi, cnt0, cnt1, z, W_post, b_post)


def kernel(x_src, x_dst, edge_index, W_src, b_src, W_dst, b_dst, W_l, W_r,
           b_sage, W_post, b_post):
    npad_e = CHP * K - EPT
    ei = edge_index.astype(jnp.int32).reshape(2, 16, EPT)
    pad_src = jnp.zeros((16, npad_e), jnp.int32)
    pad_dst = jnp.full((16, npad_e), NPAD - 1, jnp.int32)
    src3d = jnp.concatenate([ei[0], pad_src], axis=1).reshape(16, CHP, K)
    dst3d = jnp.concatenate([ei[1], pad_dst], axis=1).reshape(16, CHP, K)
    glo, ghi, z = _embed(x_src, x_dst, W_src, b_src.reshape(1, D), W_dst,
                         b_dst.reshape(1, D), W_l, W_r, b_sage.reshape(1, D))
    zrow = jnp.zeros((128, DH), jnp.float32)
    zcnt = jnp.zeros((HR, 128), jnp.int32)
    iota = jnp.arange(HR, dtype=jnp.int32)
    slo, shi, cnt0, cnt1 = _sc_segsum(glo, ghi, src3d, dst3d, zrow, zcnt,
                                      iota)
    c0 = cnt0.reshape(NPAD, 1)
    c1 = cnt1.reshape(NPAD, 1)
    return _post(slo, shi, c0, c1, z, W_post, b_post.reshape(1, D))


# speedup vs baseline: 1.1061x; 1.0897x over previous
import functools

import jax
import jax.numpy as jnp
from jax import lax
from jax.experimental import pallas as pl
from jax.experimental.pallas import tpu as pltpu
from jax.experimental.pallas import tpu_sc as plsc

N_SRC = 10000
N_DST = 10000
E = 160000
D = 256
DH = 128

K = 128
EPT = E // 16
CHP = 80
SB = 16
NB = 2
NPAD = 10240
RPT = NPAD // 16
HR = NPAD // 128


def _gelu(x):
    return 0.5 * x * (1.0 + lax.erf(x * 0.7071067811865476))


def _dot_t(x, w):
    return lax.dot_general(x, w, (((1,), (1,)), ((), ())),
                           preferred_element_type=jnp.float32)


def _embed_body(xs_ref, xd_ref, ws_ref, bs_ref, wd_ref, bd_ref, wl_ref,
                wr_ref, bsage_ref, glo_ref, ghi_ref, z_ref):
    h_src = _gelu(_dot_t(xs_ref[...], ws_ref[...]) + bs_ref[...])
    g = _dot_t(h_src, wl_ref[...])
    glo_ref[...] = g[:, :DH]
    ghi_ref[...] = g[:, DH:]
    h_dst = _gelu(_dot_t(xd_ref[...], wd_ref[...]) + bd_ref[...])
    z_ref[...] = _dot_t(h_dst, wr_ref[...]) + bsage_ref[...]


_EMBED_R = 2000


def _embed(x_src, x_dst, W_src, b_src, W_dst, b_dst, W_l, W_r, b_sage):
    n = N_SRC // _EMBED_R
    row = lambda i: (i, 0)
    full = lambda i: (0, 0)
    gh = jax.ShapeDtypeStruct((N_SRC, DH), jnp.float32)
    return pl.pallas_call(
        _embed_body,
        grid=(n,),
        in_specs=[
            pl.BlockSpec((_EMBED_R, D), row),
            pl.BlockSpec((_EMBED_R, D), row),
            pl.BlockSpec((D, D), full),
            pl.BlockSpec((1, D), full),
            pl.BlockSpec((D, D), full),
            pl.BlockSpec((1, D), full),
            pl.BlockSpec((D, D), full),
            pl.BlockSpec((D, D), full),
            pl.BlockSpec((1, D), full),
        ],
        out_specs=[
            pl.BlockSpec((_EMBED_R, DH), row),
            pl.BlockSpec((_EMBED_R, DH), row),
            pl.BlockSpec((_EMBED_R, D), row),
        ],
        out_shape=[gh, gh, jax.ShapeDtypeStruct((N_DST, D), jnp.float32)],
    )(x_src, x_dst, W_src, b_src, W_dst, b_dst, W_l, W_r, b_sage)


def _sc_body(g_lo, g_hi, src3d, dst3d, zrow_hbm, zcnt_hbm, iota_hbm,
             out_lo, out_hi, ocnt0, ocnt1, svm, dvm, gbufs, hist,
             iota_v, acc, cntS, sems_g, sems_s):
    c = lax.axis_index("c")
    s = lax.axis_index("s")
    row0 = s * RPT

    for i in range(RPT // 128):
        pltpu.sync_copy(zrow_hbm, acc.at[pl.ds(row0 + i * 128, 128)])
    pltpu.sync_copy(zcnt_hbm, hist)

    @pl.when(s == 0)
    def _():
        pltpu.sync_copy(zcnt_hbm, cntS)

    pltpu.sync_copy(iota_hbm, iota_v)
    plsc.subcore_barrier()

    ones_i = jnp.full((16,), 1, jnp.int32)

    def gather(jj):
        @pl.when(c == 0)
        def _():
            pltpu.async_copy(g_lo.at[svm.at[jj]], gbufs[jj % NB],
                             sems_g[jj % NB])

        @pl.when(c == 1)
        def _():
            pltpu.async_copy(g_hi.at[svm.at[jj]], gbufs[jj % NB],
                             sems_g[jj % NB])

    def gwait(jj):
        pltpu.make_async_copy(g_lo.at[svm.at[jj]], gbufs[jj % NB],
                              sems_g[jj % NB]).wait()

    def swait(jj):
        pltpu.make_async_copy(gbufs[jj % NB], acc.at[dvm.at[jj]],
                              sems_s[jj % NB]).wait()

    def block(b, carry):
        pltpu.sync_copy(src3d.at[s, pl.ds(b * SB, SB)], svm)
        pltpu.sync_copy(dst3d.at[s, pl.ds(b * SB, SB)], dvm)
        gather(0)
        for jj in range(SB):
            if jj + 1 < SB:
                gather(jj + 1)
            gwait(jj)
            pltpu.async_copy(gbufs[jj % NB], acc.at[dvm.at[jj]],
                             sems_s[jj % NB], add=True)

            @pl.when(c == (jj % 2))
            def _():
                for i in range(K // 16):
                    d16 = dvm[jj, pl.ds(i * 16, 16)]
                    plsc.addupdate_scatter(
                        hist,
                        [lax.shift_right_logical(d16, 7),
                         lax.bitwise_and(d16, 127)],
                        ones_i)

            swait(jj)

        return carry

    lax.fori_loop(0, CHP // SB, block, 0)
    plsc.subcore_barrier()

    pltpu.sync_copy(hist, cntS.at[iota_v], add=True)
    plsc.subcore_barrier()

    @pl.when(c == 0)
    def _():
        pltpu.sync_copy(acc.at[pl.ds(row0, RPT)], out_lo.at[pl.ds(row0, RPT)])

        @pl.when(s == 0)
        def _():
            pltpu.sync_copy(cntS, ocnt0)

    @pl.when(c == 1)
    def _():
        pltpu.sync_copy(acc.at[pl.ds(row0, RPT)], out_hi.at[pl.ds(row0, RPT)])

        @pl.when(s == 0)
        def _():
            pltpu.sync_copy(cntS, ocnt1)


_sc_segsum = functools.partial(
    pl.kernel,
    out_type=[
        jax.ShapeDtypeStruct((NPAD, DH), jnp.float32),
        jax.ShapeDtypeStruct((NPAD, DH), jnp.float32),
        jax.ShapeDtypeStruct((HR, 128), jnp.int32),
        jax.ShapeDtypeStruct((HR, 128), jnp.int32),
    ],
    mesh=plsc.VectorSubcoreMesh(core_axis_name="c", subcore_axis_name="s"),
    compiler_params=pltpu.CompilerParams(needs_layout_passes=False),
    scratch_types=[
        pltpu.VMEM((SB, K), jnp.int32),
        pltpu.VMEM((SB, K), jnp.int32),
        [pltpu.VMEM((K, DH), jnp.float32)] * NB,
        pltpu.VMEM((HR, 128), jnp.int32),
        pltpu.VMEM((HR,), jnp.int32),
        pltpu.VMEM_SHARED((NPAD, DH), jnp.float32),
        pltpu.VMEM_SHARED((HR, 128), jnp.int32),
        [pltpu.SemaphoreType.DMA] * NB,
        [pltpu.SemaphoreType.DMA] * NB,
    ],
)(_sc_body)


def _post_body(slo_ref, shi_ref, c0_ref, c1_ref, z_ref, wp_ref, bp_ref,
               out_ref):
    cnt = (c0_ref[...] + c1_ref[...]).astype(jnp.float32)
    r = 1.0 / jnp.maximum(cnt, 1.0)
    agg = jnp.concatenate([slo_ref[...], shi_ref[...]], axis=1) * r
    t = agg + z_ref[...]
    out_ref[...] = _gelu(_dot_t(t, wp_ref[...]) + bp_ref[...])


def _post(slo, shi, cnt0, cnt1, z, W_post, b_post):
    n = N_DST // _EMBED_R
    row = lambda i: (i, 0)
    full = lambda i: (0, 0)
    return pl.pallas_call(
        _post_body,
        grid=(n,),
        in_specs=[
            pl.BlockSpec((_EMBED_R, DH), row),
            pl.BlockSpec((_EMBED_R, DH), row),
            pl.BlockSpec((_EMBED_R, 1), row),
            pl.BlockSpec((_EMBED_R, 1), row),
            pl.BlockSpec((_EMBED_R, D), row),
            pl.BlockSpec((D, D), full),
            pl.BlockSpec((1, D), full),
        ],
        out_specs=pl.BlockSpec((_EMBED_R, D), row),
        out_shape=jax.ShapeDtypeStruct((N_DST, D), jnp.float32),
    )(slo, shi, cnt0, cnt1, z, W_post, b_post)


def kernel(x_src, x_dst, edge_index, W_src, b_src, W_dst, b_dst, W_l, W_r,
           b_sage, W_post, b_post):
    npad_e = CHP * K - EPT
    ei = edge_index.astype(jnp.int32).reshape(2, 16, EPT)
    pad_src = jnp.zeros((16, npad_e), jnp.int32)
    pad_dst = jnp.full((16, npad_e), NPAD - 1, jnp.int32)
    src3d = jnp.concatenate([ei[0], pad_src], axis=1).reshape(16, CHP, K)
    dst3d = jnp.concatenate([ei[1], pad_dst], axis=1).reshape(16, CHP, K)
    glo, ghi, z = _embed(x_src, x_dst, W_src, b_src.reshape(1, D), W_dst,
                         b_dst.reshape(1, D), W_l, W_r, b_sage.reshape(1, D))
    zrow = jnp.zeros((128, DH), jnp.float32)
    zcnt = jnp.zeros((HR, 128), jnp.int32)
    iota = jnp.arange(HR, dtype=jnp.int32)
    slo, shi, cnt0, cnt1 = _sc_segsum(glo, ghi, src3d, dst3d, zrow, zcnt,
                                      iota)
    c0 = cnt0.reshape(NPAD, 1)
    c1 = cnt1.reshape(NPAD, 1)
    return _post(slo, shi, c0, c1, z, W_post, b_post.reshape(1, D))


# speedup vs baseline: 1.1124x; 1.0057x over previous
import functools

import jax
import jax.numpy as jnp
from jax import lax
from jax.experimental import pallas as pl
from jax.experimental.pallas import tpu as pltpu
from jax.experimental.pallas import tpu_sc as plsc

N_SRC = 10000
N_DST = 10000
E = 160000
D = 256
DH = 128

K = 128
EPT = E // 16
CHP = 80
SB = 16
NB = 2
NPAD = 10240
RPT = NPAD // 16
HR = NPAD // 128


def _gelu(x):
    return 0.5 * x * (1.0 + lax.erf(x * 0.7071067811865476))


def _dot_t(x, w):
    return lax.dot_general(x, w, (((1,), (1,)), ((), ())),
                           preferred_element_type=jnp.float32)


def _embed_body(xs_ref, xd_ref, ws_ref, bs_ref, wd_ref, bd_ref, wl_ref,
                wr_ref, bsage_ref, glo_ref, ghi_ref, z_ref):
    h_src = _gelu(_dot_t(xs_ref[...], ws_ref[...]) + bs_ref[...])
    g = _dot_t(h_src, wl_ref[...])
    glo_ref[...] = g[:, :DH]
    ghi_ref[...] = g[:, DH:]
    h_dst = _gelu(_dot_t(xd_ref[...], wd_ref[...]) + bd_ref[...])
    z_ref[...] = _dot_t(h_dst, wr_ref[...]) + bsage_ref[...]


_EMBED_R = 2000


def _embed(x_src, x_dst, W_src, b_src, W_dst, b_dst, W_l, W_r, b_sage):
    n = N_SRC // _EMBED_R
    row = lambda i: (i, 0)
    full = lambda i: (0, 0)
    gh = jax.ShapeDtypeStruct((N_SRC, DH), jnp.float32)
    return pl.pallas_call(
        _embed_body,
        grid=(n,),
        in_specs=[
            pl.BlockSpec((_EMBED_R, D), row),
            pl.BlockSpec((_EMBED_R, D), row),
            pl.BlockSpec((D, D), full),
            pl.BlockSpec((1, D), full),
            pl.BlockSpec((D, D), full),
            pl.BlockSpec((1, D), full),
            pl.BlockSpec((D, D), full),
            pl.BlockSpec((D, D), full),
            pl.BlockSpec((1, D), full),
        ],
        out_specs=[
            pl.BlockSpec((_EMBED_R, DH), row),
            pl.BlockSpec((_EMBED_R, DH), row),
            pl.BlockSpec((_EMBED_R, D), row),
        ],
        out_shape=[gh, gh, jax.ShapeDtypeStruct((N_DST, D), jnp.float32)],
    )(x_src, x_dst, W_src, b_src, W_dst, b_dst, W_l, W_r, b_sage)


def _sc_body(g_lo, g_hi, src3d, dst3d, zrow_hbm, zcnt_hbm, iota_hbm,
             out_lo, out_hi, ocnt0, ocnt1, svm, dvm, gbufs, hist,
             iota_v, acc, cntS, sems_g, sems_s):
    c = lax.axis_index("c")
    s = lax.axis_index("s")
    row0 = s * RPT

    zdescs = []
    for i in range(RPT // 128):
        zdescs.append(pltpu.async_copy(
            zrow_hbm, acc.at[pl.ds(row0 + i * 128, 128)], sems_g[0]))
    zdescs.append(pltpu.async_copy(zcnt_hbm, hist, sems_g[1]))
    zdescs.append(pltpu.async_copy(iota_hbm, iota_v, sems_s[0]))

    @pl.when(s == 0)
    def _():
        pltpu.async_copy(zcnt_hbm, cntS, sems_s[1]).wait()

    for d in zdescs:
        d.wait()
    plsc.subcore_barrier()

    ones_i = jnp.full((16,), 1, jnp.int32)

    def gather(jj):
        @pl.when(c == 0)
        def _():
            pltpu.async_copy(g_lo.at[svm.at[jj]], gbufs[jj % NB],
                             sems_g[jj % NB])

        @pl.when(c == 1)
        def _():
            pltpu.async_copy(g_hi.at[svm.at[jj]], gbufs[jj % NB],
                             sems_g[jj % NB])

    def gwait(jj):
        pltpu.make_async_copy(g_lo.at[svm.at[jj]], gbufs[jj % NB],
                              sems_g[jj % NB]).wait()

    def swait(jj):
        pltpu.make_async_copy(gbufs[jj % NB], acc.at[dvm.at[jj]],
                              sems_s[jj % NB]).wait()

    def block(b, carry):
        pltpu.sync_copy(src3d.at[s, pl.ds(b * SB, SB)], svm)
        pltpu.sync_copy(dst3d.at[s, pl.ds(b * SB, SB)], dvm)
        gather(0)
        for jj in range(SB):
            if jj + 1 < SB:
                gather(jj + 1)
            gwait(jj)
            pltpu.async_copy(gbufs[jj % NB], acc.at[dvm.at[jj]],
                             sems_s[jj % NB], add=True)

            @pl.when(c == (jj % 2))
            def _():
                for i in range(K // 16):
                    d16 = dvm[jj, pl.ds(i * 16, 16)]
                    plsc.addupdate_scatter(
                        hist,
                        [lax.shift_right_logical(d16, 7),
                         lax.bitwise_and(d16, 127)],
                        ones_i)

            swait(jj)

        return carry

    lax.fori_loop(0, CHP // SB, block, 0)
    plsc.subcore_barrier()

    pltpu.sync_copy(hist, cntS.at[iota_v], add=True)
    plsc.subcore_barrier()

    @pl.when(c == 0)
    def _():
        pltpu.sync_copy(acc.at[pl.ds(row0, RPT)], out_lo.at[pl.ds(row0, RPT)])

        @pl.when(s == 0)
        def _():
            pltpu.sync_copy(cntS, ocnt0)

    @pl.when(c == 1)
    def _():
        pltpu.sync_copy(acc.at[pl.ds(row0, RPT)], out_hi.at[pl.ds(row0, RPT)])

        @pl.when(s == 0)
        def _():
            pltpu.sync_copy(cntS, ocnt1)


_sc_segsum = functools.partial(
    pl.kernel,
    out_type=[
        jax.ShapeDtypeStruct((NPAD, DH), jnp.float32),
        jax.ShapeDtypeStruct((NPAD, DH), jnp.float32),
        jax.ShapeDtypeStruct((HR, 128), jnp.int32),
        jax.ShapeDtypeStruct((HR, 128), jnp.int32),
    ],
    mesh=plsc.VectorSubcoreMesh(core_axis_name="c", subcore_axis_name="s"),
    compiler_params=pltpu.CompilerParams(needs_layout_passes=False),
    scratch_types=[
        pltpu.VMEM((SB, K), jnp.int32),
        pltpu.VMEM((SB, K), jnp.int32),
        [pltpu.VMEM((K, DH), jnp.float32)] * NB,
        pltpu.VMEM((HR, 128), jnp.int32),
        pltpu.VMEM((HR,), jnp.int32),
        pltpu.VMEM_SHARED((NPAD, DH), jnp.float32),
        pltpu.VMEM_SHARED((HR, 128), jnp.int32),
        [pltpu.SemaphoreType.DMA] * NB,
        [pltpu.SemaphoreType.DMA] * NB,
    ],
)(_sc_body)


def _post_body(slo_ref, shi_ref, c0_ref, c1_ref, z_ref, wp_ref, bp_ref,
               out_ref):
    cnt = (c0_ref[...] + c1_ref[...]).astype(jnp.float32)
    r = 1.0 / jnp.maximum(cnt, 1.0)
    agg = jnp.concatenate([slo_ref[...], shi_ref[...]], axis=1) * r
    t = agg + z_ref[...]
    out_ref[...] = _gelu(_dot_t(t, wp_ref[...]) + bp_ref[...])


def _post(slo, shi, cnt0, cnt1, z, W_post, b_post):
    n = N_DST // _EMBED_R
    row = lambda i: (i, 0)
    full = lambda i: (0, 0)
    return pl.pallas_call(
        _post_body,
        grid=(n,),
        in_specs=[
            pl.BlockSpec((_EMBED_R, DH), row),
            pl.BlockSpec((_EMBED_R, DH), row),
            pl.BlockSpec((_EMBED_R, 1), row),
            pl.BlockSpec((_EMBED_R, 1), row),
            pl.BlockSpec((_EMBED_R, D), row),
            pl.BlockSpec((D, D), full),
            pl.BlockSpec((1, D), full),
        ],
        out_specs=pl.BlockSpec((_EMBED_R, D), row),
        out_shape=jax.ShapeDtypeStruct((N_DST, D), jnp.float32),
    )(slo, shi, cnt0, cnt1, z, W_post, b_post)


def kernel(x_src, x_dst, edge_index, W_src, b_src, W_dst, b_dst, W_l, W_r,
           b_sage, W_post, b_post):
    npad_e = CHP * K - EPT
    ei = edge_index.astype(jnp.int32).reshape(2, 16, EPT)
    pad_src = jnp.zeros((16, npad_e), jnp.int32)
    pad_dst = jnp.full((16, npad_e), NPAD - 1, jnp.int32)
    src3d = jnp.concatenate([ei[0], pad_src], axis=1).reshape(16, CHP, K)
    dst3d = jnp.concatenate([ei[1], pad_dst], axis=1).reshape(16, CHP, K)
    glo, ghi, z = _embed(x_src, x_dst, W_src, b_src.reshape(1, D), W_dst,
                         b_dst.reshape(1, D), W_l, W_r, b_sage.reshape(1, D))
    zrow = jnp.zeros((128, DH), jnp.float32)
    zcnt = jnp.zeros((HR, 128), jnp.int32)
    iota = jnp.arange(HR, dtype=jnp.int32)
    slo, shi, cnt0, cnt1 = _sc_segsum(glo, ghi, src3d, dst3d, zrow, zcnt,
                                      iota)
    c0 = cnt0.reshape(NPAD, 1)
    c1 = cnt1.reshape(NPAD, 1)
    return _post(slo, shi, c0, c1, z, W_post, b_post.reshape(1, D))
